# baseline (speedup 1.0000x reference)
import jax
import jax.numpy as jnp
from jax.experimental import pallas as pl
from jax.experimental.pallas import tpu as pltpu


def _model_kernel(idx_ref, gate_ref, x_ref, we_ref, be_ref, wq_ref, wk_ref,
                  wproj_ref, hw_ref, hb_ref, out_ref, bal_ref, imp_ref):
    b = pl.program_id(0)
    nb = pl.num_programs(0)
    E = we_ref.shape[0]

    x = x_ref[0]
    mean = jnp.mean(x, axis=0, keepdims=True)
    var = jnp.mean((x - mean) ** 2, axis=0, keepdims=True)
    std = jnp.sqrt(var + 1e-5)
    xn = (x - mean) / std

    i0 = idx_ref[b, 0]
    i1 = idx_ref[b, 1]
    g0 = gate_ref[b, 0]
    g1 = gate_ref[b, 1]

    wc = g0 * we_ref[i0] + g1 * we_ref[i1]
    bias = g0 * be_ref[pl.ds(i0, 1), :] + g1 * be_ref[pl.ds(i1, 1), :]

    dn0 = (((0,), (0,)), ((), ()))
    moe = jax.lax.dot_general(xn, wc, dn0, preferred_element_type=jnp.float32)
    moe = moe + bias

    q = jax.lax.dot_general(xn, wq_ref[...], dn0, preferred_element_type=jnp.float32)
    k = jax.lax.dot_general(xn, wk_ref[...], dn0, preferred_element_type=jnp.float32)
    s = jax.lax.dot_general(q, k, (((1,), (1,)), ((), ())),
                            preferred_element_type=jnp.float32) * 0.125
    s = s - jnp.max(s, axis=1, keepdims=True)
    es = jnp.exp(s)
    attn = es / jnp.sum(es, axis=1, keepdims=True)
    proj = jax.lax.dot_general(xn, wproj_ref[...], dn0,
                               preferred_element_type=jnp.float32)
    vout = jnp.dot(attn, proj, preferred_element_type=jnp.float32)

    dec = jnp.concatenate([moe, vout], axis=1)
    outD = jnp.dot(dec, hw_ref[...], preferred_element_type=jnp.float32)
    outD = outD + hb_ref[...]
    out_ref[0] = outD.T * std + mean

    @pl.when(b == 0)
    def _():
        for e in range(E):
            imp_ref[e] = 0.0

    imp_ref[i0] = imp_ref[i0] + g0
    imp_ref[i1] = imp_ref[i1] + g1

    @pl.when(b == nb - 1)
    def _():
        m = 0.0
        for e in range(E):
            m = m + imp_ref[e]
        m = m / E
        v = 0.0
        for e in range(E):
            v = v + (imp_ref[e] - m) ** 2
        v = v / E
        bal_ref[0] = v / (m * m + 1e-10)


def kernel(x_enc, x_mark_enc, x_dec, x_mark_dec, w_gate, w_noise, W_experts,
           b_experts, Wq, Wk, Wproj, head_W, head_b, *, interpret=False):
    B, T, C, F = x_enc.shape
    D = C * F
    E, _, P = W_experts.shape
    dk = Wq.shape[-1]
    eps = 1e-5

    mean = jnp.mean(x_enc, axis=1, keepdims=True)
    std = jnp.sqrt(jnp.var(x_enc, axis=1, keepdims=True) + eps)
    xn = (x_enc - mean) / std
    x2 = xn.reshape(B, T, D)
    feats = x2.mean(axis=1)
    clean_logits = feats @ w_gate
    top_vals, top_idx = jax.lax.top_k(clean_logits, 2)
    top_gates = jax.nn.softmax(top_vals, axis=-1)

    x_flat = x_enc.reshape(B, T, D)
    out, bal = pl.pallas_call(
        _model_kernel,
        grid=(B,),
        in_specs=[
            pl.BlockSpec(memory_space=pltpu.SMEM),
            pl.BlockSpec(memory_space=pltpu.SMEM),
            pl.BlockSpec((1, T, D), lambda b: (b, 0, 0)),
            pl.BlockSpec((E, T, P), lambda b: (0, 0, 0)),
            pl.BlockSpec((E, P), lambda b: (0, 0)),
            pl.BlockSpec((T, dk), lambda b: (0, 0)),
            pl.BlockSpec((T, dk), lambda b: (0, 0)),
            pl.BlockSpec((T, P), lambda b: (0, 0)),
            pl.BlockSpec((2 * P, P), lambda b: (0, 0)),
            pl.BlockSpec((1, P), lambda b: (0, 0)),
        ],
        out_specs=[
            pl.BlockSpec((1, P, D), lambda b: (b, 0, 0)),
            pl.BlockSpec(memory_space=pltpu.SMEM),
        ],
        out_shape=[
            jax.ShapeDtypeStruct((B, P, D), jnp.float32),
            jax.ShapeDtypeStruct((1,), jnp.float32),
        ],
        scratch_shapes=[pltpu.SMEM((E,), jnp.float32)],
        interpret=interpret,
    )(top_idx, top_gates, x_flat, W_experts, b_experts, Wq, Wk, Wproj,
      head_W, head_b.reshape(1, P))

    return out.reshape(B, P, C), bal[0]


# speedup vs baseline: 2.6172x; 2.6172x over previous
import jax
import jax.numpy as jnp
from jax.experimental import pallas as pl
from jax.experimental.pallas import tpu as pltpu


def _model_kernel(idx_ref, gate_ref, xt_ref, w3_ref, we_ref, be_ref, hw_ref,
                  hb_ref, mean_ref, std_ref, out_ref, bal_ref, imp_ref):
    b = pl.program_id(0)
    nb = pl.num_programs(0)
    E = we_ref.shape[0]
    dk = 64
    P = we_ref.shape[2]

    i0 = idx_ref[b, 0]
    i1 = idx_ref[b, 1]
    g0 = gate_ref[b, 0]
    g1 = gate_ref[b, 1]

    xt = xt_ref[0]

    wc = (g0 * we_ref[i0].astype(jnp.float32)
          + g1 * we_ref[i1].astype(jnp.float32)).astype(jnp.bfloat16)
    wf = jnp.concatenate([w3_ref[...], wc], axis=1)

    fused = jax.lax.dot_general(xt, wf, (((1,), (0,)), ((), ())),
                                preferred_element_type=jnp.float32)
    q = fused[:, :dk]
    k = fused[:, dk:2 * dk]
    proj = fused[:, 2 * dk:2 * dk + P]
    moe = fused[:, 2 * dk + P:]
    bias = g0 * be_ref[pl.ds(i0, 1), :] + g1 * be_ref[pl.ds(i1, 1), :]
    moe = moe + bias

    s = jax.lax.dot_general(q.astype(jnp.bfloat16), k.astype(jnp.bfloat16),
                            (((1,), (1,)), ((), ())),
                            preferred_element_type=jnp.float32) * 0.125
    s = s - jnp.max(s, axis=1, keepdims=True)
    es = jnp.exp(s)
    attn = es / jnp.sum(es, axis=1, keepdims=True)

    vout = jnp.dot(attn.astype(jnp.bfloat16), proj.astype(jnp.bfloat16),
                   preferred_element_type=jnp.float32)

    dec = jnp.concatenate([moe, vout], axis=1).astype(jnp.bfloat16)
    outD = jnp.dot(dec, hw_ref[...], preferred_element_type=jnp.float32)
    outD = outD + hb_ref[...]
    out_ref[0] = outD * std_ref[0] + mean_ref[0]

    @pl.when(b == 0)
    def _():
        for e in range(E):
            imp_ref[e] = 0.0

    imp_ref[i0] = imp_ref[i0] + g0
    imp_ref[i1] = imp_ref[i1] + g1

    @pl.when(b == nb - 1)
    def _():
        m = 0.0
        for e in range(E):
            m = m + imp_ref[e]
        m = m / E
        v = 0.0
        for e in range(E):
            v = v + (imp_ref[e] - m) ** 2
        v = v / E
        bal_ref[0] = v / (m * m + 1e-10)


def kernel(x_enc, x_mark_enc, x_dec, x_mark_dec, w_gate, w_noise, W_experts,
           b_experts, Wq, Wk, Wproj, head_W, head_b, *, interpret=False):
    B, T, C, F = x_enc.shape
    D = C * F
    E, _, P = W_experts.shape
    dk = Wq.shape[-1]
    eps = 1e-5

    mean = jnp.mean(x_enc, axis=1, keepdims=True)
    std = jnp.sqrt(jnp.var(x_enc, axis=1, keepdims=True) + eps)
    xn = (x_enc - mean) / std
    x2 = xn.reshape(B, T, D)
    feats = x2.mean(axis=1)
    clean_logits = feats @ w_gate
    top_vals, top_idx = jax.lax.top_k(clean_logits, 2)
    top_gates = jax.nn.softmax(top_vals, axis=-1)

    xt = jnp.swapaxes(x2, 1, 2).astype(jnp.bfloat16)
    mean_d = mean.reshape(B, D)[:, :, None]
    std_d = std.reshape(B, D)[:, :, None]
    w3 = jnp.concatenate([Wq, Wk, Wproj], axis=1).astype(jnp.bfloat16)
    we = W_experts.astype(jnp.bfloat16)
    hw = head_W.astype(jnp.bfloat16)

    out, bal = pl.pallas_call(
        _model_kernel,
        grid=(B,),
        in_specs=[
            pl.BlockSpec(memory_space=pltpu.SMEM),
            pl.BlockSpec(memory_space=pltpu.SMEM),
            pl.BlockSpec((1, D, T), lambda b: (b, 0, 0)),
            pl.BlockSpec((T, 2 * dk + P), lambda b: (0, 0)),
            pl.BlockSpec((E, T, P), lambda b: (0, 0, 0)),
            pl.BlockSpec((E, P), lambda b: (0, 0)),
            pl.BlockSpec((2 * P, P), lambda b: (0, 0)),
            pl.BlockSpec((1, P), lambda b: (0, 0)),
            pl.BlockSpec((1, D, 1), lambda b: (b, 0, 0)),
            pl.BlockSpec((1, D, 1), lambda b: (b, 0, 0)),
        ],
        out_specs=[
            pl.BlockSpec((1, D, P), lambda b: (b, 0, 0)),
            pl.BlockSpec(memory_space=pltpu.SMEM),
        ],
        out_shape=[
            jax.ShapeDtypeStruct((B, D, P), jnp.float32),
            jax.ShapeDtypeStruct((1,), jnp.float32),
        ],
        scratch_shapes=[pltpu.SMEM((E,), jnp.float32)],
        interpret=interpret,
    )(top_idx, top_gates, xt, w3, we, b_experts, hw, head_b.reshape(1, P),
      mean_d, std_d)

    return jnp.swapaxes(out, 1, 2).reshape(B, P, C), bal[0]
